# baseline (speedup 1.0000x reference)
import functools

import jax
import jax.numpy as jnp
from jax import lax
from jax.experimental import pallas as pl
from jax.experimental.pallas import tpu as pltpu
from jax.experimental.pallas import tpu_sc as plsc

N_TASKS = 100000
N_EXPERTS = 64
BATCH = 16384

_NC = 2
_NS = 16
_NW = _NC * _NS
_E_PER_W = N_EXPERTS // _NW
_LANES = 16
_OUT_CHUNK = 4096
_N_CHUNKS = BATCH // _OUT_CHUNK

_sc_mesh = plsc.VectorSubcoreMesh(core_axis_name="c", subcore_axis_name="s")


@functools.partial(
    pl.kernel,
    mesh=_sc_mesh,
    out_type=jax.ShapeDtypeStruct((N_EXPERTS, BATCH), jnp.float32),
    scratch_types=[
        pltpu.VMEM((N_TASKS,), jnp.float32),
        pltpu.VMEM((BATCH,), jnp.int32),
        pltpu.VMEM((2, _OUT_CHUNK), jnp.float32),
        pltpu.VMEM_SHARED((BATCH,), jnp.int32),
        pltpu.SemaphoreType.DMA,
        pltpu.SemaphoreType.DMA,
    ],
    compiler_params=pltpu.CompilerParams(needs_layout_passes=False),
)
def _sc_gather(w_hbm, idx_hbm, out_hbm, row_v, idx_v, out_v, idx_sh, sem0, sem1):
    cid = lax.axis_index("c")
    sid = lax.axis_index("s")
    wid = sid * _NC + cid

    row_cp = pltpu.make_async_copy(w_hbm.at[wid * _E_PER_W], row_v, sem0)
    row_cp.start()

    @pl.when(sid == 0)
    def _():
        pltpu.sync_copy(idx_hbm, idx_sh)

    plsc.subcore_barrier()
    pltpu.sync_copy(idx_sh, idx_v)
    row_cp.wait()

    sems = (sem0, sem1)
    for ei in range(_E_PER_W):
        e = wid * _E_PER_W + ei
        if ei > 0:
            pltpu.sync_copy(w_hbm.at[e], row_v)
        for c in range(_N_CHUNKS):
            base = c * (_OUT_CHUNK // _LANES)
            buf = c % 2
            if ei * _N_CHUNKS + c >= 2:
                pltpu.make_async_copy(
                    out_v.at[buf],
                    out_hbm.at[e, pl.ds(0, _OUT_CHUNK)],
                    sems[buf],
                ).wait()

            @plsc.parallel_loop(base, base + _OUT_CHUNK // _LANES, unroll=16)
            def gather_body(i):
                ids = idx_v[pl.ds(i * _LANES, _LANES)]
                out_v[buf, pl.ds((i - base) * _LANES, _LANES)] = (
                    plsc.load_gather(row_v, [ids]))

            pltpu.async_copy(
                out_v.at[buf],
                out_hbm.at[e, pl.ds(c * _OUT_CHUNK, _OUT_CHUNK)],
                sems[buf],
            )
    for buf in range(2):
        pltpu.make_async_copy(
            out_v.at[buf],
            out_hbm.at[N_EXPERTS - 1, pl.ds(0, _OUT_CHUNK)],
            sems[buf],
        ).wait()


_TC_BLOCK = 8192


def _softmax_body(lt_ref, out_ref):
    x = lt_ref[...]
    e = jnp.exp(x)
    s = jnp.sum(e, axis=0, keepdims=True)
    out_ref[...] = e * (1.0 / s)


_tc_softmax = pl.pallas_call(
    _softmax_body,
    grid=(BATCH // _TC_BLOCK,),
    in_specs=[pl.BlockSpec((N_EXPERTS, _TC_BLOCK), lambda i: (0, i))],
    out_specs=pl.BlockSpec((N_EXPERTS, _TC_BLOCK), lambda i: (0, i)),
    out_shape=jax.ShapeDtypeStruct((N_EXPERTS, BATCH), jnp.float32),
    compiler_params=pltpu.CompilerParams(
        dimension_semantics=("parallel",)),
)


def kernel(task_idx, weight):
    logits_t = _sc_gather(weight, task_idx.astype(jnp.int32))
    return _tc_softmax(logits_t).T


# speedup vs baseline: 1.0164x; 1.0164x over previous
import functools

import jax
import jax.numpy as jnp
from jax import lax
from jax.experimental import pallas as pl
from jax.experimental.pallas import tpu as pltpu
from jax.experimental.pallas import tpu_sc as plsc

N_TASKS = 100000
N_EXPERTS = 64
BATCH = 16384

_NC = 2
_NS = 16
_NW = _NC * _NS
_E_PER_W = N_EXPERTS // _NW
_LANES = 16
_OUT_CHUNK = 4096
_N_CHUNKS = BATCH // _OUT_CHUNK

_sc_mesh = plsc.VectorSubcoreMesh(core_axis_name="c", subcore_axis_name="s")


@functools.partial(
    pl.kernel,
    mesh=_sc_mesh,
    out_type=jax.ShapeDtypeStruct((N_EXPERTS, BATCH), jnp.float32),
    scratch_types=[
        pltpu.VMEM((N_TASKS,), jnp.float32),
        pltpu.VMEM((BATCH,), jnp.int32),
        pltpu.VMEM((2, _OUT_CHUNK), jnp.float32),
        pltpu.VMEM_SHARED((BATCH,), jnp.int32),
        pltpu.SemaphoreType.DMA,
        pltpu.SemaphoreType.DMA,
    ],
    compiler_params=pltpu.CompilerParams(needs_layout_passes=False),
)
def _sc_gather(w_hbm, idx_hbm, out_hbm, row_v, idx_v, out_v, idx_sh, sem0, sem1):
    cid = lax.axis_index("c")
    sid = lax.axis_index("s")
    wid = sid * _NC + cid

    row_cp = pltpu.make_async_copy(w_hbm.at[wid * _E_PER_W], row_v, sem0)
    row_cp.start()

    @pl.when(sid == 0)
    def _():
        pltpu.sync_copy(idx_hbm, idx_sh)

    plsc.subcore_barrier()
    pltpu.sync_copy(idx_sh, idx_v)
    row_cp.wait()

    sems = (sem0, sem1)
    for ei in range(_E_PER_W):
        e = wid * _E_PER_W + ei
        if ei > 0:
            pltpu.sync_copy(w_hbm.at[e], row_v)
        for c in range(_N_CHUNKS):
            base = c * (_OUT_CHUNK // _LANES)
            buf = c % 2
            if ei * _N_CHUNKS + c >= 2:
                pltpu.make_async_copy(
                    out_v.at[buf],
                    out_hbm.at[e, pl.ds(0, _OUT_CHUNK)],
                    sems[buf],
                ).wait()

            @plsc.parallel_loop(base, base + _OUT_CHUNK // _LANES, unroll=8)
            def gather_body(i):
                ids = idx_v[pl.ds(i * _LANES, _LANES)]
                out_v[buf, pl.ds((i - base) * _LANES, _LANES)] = (
                    plsc.load_gather(row_v, [ids]))

            pltpu.async_copy(
                out_v.at[buf],
                out_hbm.at[e, pl.ds(c * _OUT_CHUNK, _OUT_CHUNK)],
                sems[buf],
            )
    for buf in range(2):
        pltpu.make_async_copy(
            out_v.at[buf],
            out_hbm.at[N_EXPERTS - 1, pl.ds(0, _OUT_CHUNK)],
            sems[buf],
        ).wait()


_TC_BLOCK = 8192


def _softmax_body(lt_ref, out_ref):
    x = lt_ref[...]
    e = jnp.exp(x)
    s = jnp.sum(e, axis=0, keepdims=True)
    out_ref[...] = e * (1.0 / s)


_tc_softmax = pl.pallas_call(
    _softmax_body,
    grid=(BATCH // _TC_BLOCK,),
    in_specs=[pl.BlockSpec((N_EXPERTS, _TC_BLOCK), lambda i: (0, i))],
    out_specs=pl.BlockSpec((N_EXPERTS, _TC_BLOCK), lambda i: (0, i)),
    out_shape=jax.ShapeDtypeStruct((N_EXPERTS, BATCH), jnp.float32),
    compiler_params=pltpu.CompilerParams(
        dimension_semantics=("parallel",)),
)


def kernel(task_idx, weight):
    logits_t = _sc_gather(weight, task_idx.astype(jnp.int32))
    return _tc_softmax(logits_t).T


# speedup vs baseline: 1.0177x; 1.0012x over previous
import functools

import jax
import jax.numpy as jnp
from jax import lax
from jax.experimental import pallas as pl
from jax.experimental.pallas import tpu as pltpu
from jax.experimental.pallas import tpu_sc as plsc

N_TASKS = 100000
N_EXPERTS = 64
BATCH = 16384

_NC = 2
_NS = 16
_NW = _NC * _NS
_E_PER_W = N_EXPERTS // _NW
_LANES = 16
_OUT_CHUNK = 4096
_N_CHUNKS = BATCH // _OUT_CHUNK

_sc_mesh = plsc.VectorSubcoreMesh(core_axis_name="c", subcore_axis_name="s")


@functools.partial(
    pl.kernel,
    mesh=_sc_mesh,
    out_type=jax.ShapeDtypeStruct((N_EXPERTS, BATCH), jnp.float32),
    scratch_types=[
        pltpu.VMEM((N_TASKS,), jnp.float32),
        pltpu.VMEM((BATCH,), jnp.int32),
        pltpu.VMEM((2, _OUT_CHUNK), jnp.float32),
        pltpu.VMEM_SHARED((BATCH,), jnp.int32),
        pltpu.SemaphoreType.DMA,
        pltpu.SemaphoreType.DMA,
        pltpu.SemaphoreType.DMA,
    ],
    compiler_params=pltpu.CompilerParams(needs_layout_passes=False),
)
def _sc_gather(w_hbm, idx_hbm, out_hbm, row_v, idx_v, out_v, idx_sh,
               sem0, sem1, sem_row):
    cid = lax.axis_index("c")
    sid = lax.axis_index("s")
    wid = sid * _NC + cid

    row_cp = pltpu.make_async_copy(w_hbm.at[wid * _E_PER_W], row_v, sem_row)
    row_cp.start()

    @pl.when(sid == 0)
    def _():
        pltpu.sync_copy(idx_hbm, idx_sh)

    plsc.subcore_barrier()
    pltpu.sync_copy(idx_sh, idx_v)
    row_cp.wait()

    sems = (sem0, sem1)
    for ei in range(_E_PER_W):
        e = wid * _E_PER_W + ei
        if ei > 0:
            pltpu.sync_copy(w_hbm.at[e], row_v)
        for c in range(_N_CHUNKS):
            base = c * (_OUT_CHUNK // _LANES)
            buf = c % 2
            if ei * _N_CHUNKS + c >= 2:
                pltpu.make_async_copy(
                    out_v.at[buf],
                    out_hbm.at[e, pl.ds(0, _OUT_CHUNK)],
                    sems[buf],
                ).wait()

            @plsc.parallel_loop(base, base + _OUT_CHUNK // _LANES, unroll=8)
            def gather_body(i):
                ids = idx_v[pl.ds(i * _LANES, _LANES)]
                out_v[buf, pl.ds((i - base) * _LANES, _LANES)] = (
                    plsc.load_gather(row_v, [ids]))

            pltpu.async_copy(
                out_v.at[buf],
                out_hbm.at[e, pl.ds(c * _OUT_CHUNK, _OUT_CHUNK)],
                sems[buf],
            )
    for buf in range(2):
        pltpu.make_async_copy(
            out_v.at[buf],
            out_hbm.at[N_EXPERTS - 1, pl.ds(0, _OUT_CHUNK)],
            sems[buf],
        ).wait()


_TC_BLOCK = 8192


def _softmax_body(lt_ref, out_ref):
    x = lt_ref[...]
    e = jnp.exp(x)
    s = jnp.sum(e, axis=0, keepdims=True)
    out_ref[...] = e * (1.0 / s)


_tc_softmax = pl.pallas_call(
    _softmax_body,
    grid=(BATCH // _TC_BLOCK,),
    in_specs=[pl.BlockSpec((N_EXPERTS, _TC_BLOCK), lambda i: (0, i))],
    out_specs=pl.BlockSpec((N_EXPERTS, _TC_BLOCK), lambda i: (0, i)),
    out_shape=jax.ShapeDtypeStruct((N_EXPERTS, BATCH), jnp.float32),
    compiler_params=pltpu.CompilerParams(
        dimension_semantics=("parallel",)),
)


def kernel(task_idx, weight):
    logits_t = _sc_gather(weight, task_idx.astype(jnp.int32))
    return _tc_softmax(logits_t).T
